# baseline (speedup 1.0000x reference)
import jax
import jax.numpy as jnp
from jax import lax
from jax.experimental import pallas as pl
from jax.experimental.pallas import tpu as pltpu
from jax.experimental.pallas import tpu_sc as plsc

N = 10000
E = 320000
D_IN = 128
D_HID = 128
D_OUT = 64

NP = 10240
RPT = NP // 16
C = 128
K = 160
EPT = C * K
EPAD = 16 * EPT

BR = 1024
NPB = NP // BR


def _make_segsum(d_half, with_deg):
  mesh = plsc.VectorSubcoreMesh(core_axis_name="c", subcore_axis_name="s")
  out_type = [jax.ShapeDtypeStruct((2 * NP, d_half), jnp.bfloat16)]
  if with_deg:
    out_type.append(jax.ShapeDtypeStruct((2 * NP,), jnp.float32))
  NB = 8
  G = 6
  scratch = [
      pltpu.VMEM((K, C), jnp.int32),
      pltpu.VMEM((K, C), jnp.int32),
      pltpu.VMEM((NB * C, d_half), jnp.bfloat16),
      pltpu.VMEM_SHARED((NP, d_half), jnp.bfloat16),
      pltpu.VMEM_SHARED((NP, d_half), jnp.bfloat16),
  ] + [pltpu.SemaphoreType.DMA] * (2 * NB)
  if with_deg:
    scratch += [
        pltpu.VMEM((C,), jnp.float32),
        pltpu.VMEM_SHARED((NP,), jnp.float32),
        pltpu.SemaphoreType.DMA,
    ]

  def body(ya_hbm, yb_hbm, src_hbm, dst_hbm, zrow_hbm, *rest):
    if with_deg:
      (zdeg_hbm, out_hbm, odeg_hbm, src_v, dst_v, ring, acc, ystage) = rest[:8]
      rest = rest[8:]
    else:
      (out_hbm, src_v, dst_v, ring, acc, ystage) = rest[:6]
      rest = rest[6:]
    gs = rest[:NB]
    ss = rest[NB:2 * NB]
    rest = rest[2 * NB:]
    if with_deg:
      ones_v, dacc, dsem = rest
    rows = [ring.at[pl.ds(b * C, C)] for b in range(NB)]
    cid = lax.axis_index("c")
    sid = lax.axis_index("s")
    row0 = pl.multiple_of(sid * RPT, 8)

    pltpu.sync_copy(src_hbm.at[sid], src_v)
    pltpu.sync_copy(dst_hbm.at[sid], dst_v)
    pltpu.sync_copy(zrow_hbm, acc.at[pl.ds(row0, RPT)])

    @pl.when(cid == 0)
    def _stage_a():
      pltpu.sync_copy(ya_hbm.at[pl.ds(row0, RPT)], ystage.at[pl.ds(row0, RPT)])

    @pl.when(cid == 1)
    def _stage_b():
      pltpu.sync_copy(yb_hbm.at[pl.ds(row0, RPT)], ystage.at[pl.ds(row0, RPT)])
    if with_deg:
      for i in range(C // 16):
        ones_v[pl.ds(i * 16, 16)] = jnp.ones((16,), jnp.float32)
      pltpu.sync_copy(zdeg_hbm, dacc.at[pl.ds(row0, RPT)])
    plsc.subcore_barrier()

    def issue_gather(k, b, even):
      if even:
        pltpu.async_copy(ystage.at[src_v.at[k]], rows[b], gs[b])
      else:
        @pl.when(cid == 0)
        def _ga():
          pltpu.async_copy(ya_hbm.at[src_v.at[k]], rows[b], gs[b])

        @pl.when(cid == 1)
        def _gb():
          pltpu.async_copy(yb_hbm.at[src_v.at[k]], rows[b], gs[b])

    for k in range(G):
      issue_gather(k, k % NB, k % 2 == 0)

    def ring(g, carry):
      k0 = g * NB
      for b in range(NB):
        k = k0 + b
        bg = (b + G) % NB

        @pl.when(k + G < K)
        def _refill():
          @pl.when(k >= NB - G)
          def _reclaim():
            pltpu.make_async_copy(rows[bg], acc.at[dst_v.at[0]],
                                  ss[bg]).wait()
          issue_gather(k + G, bg, (b + G) % 2 == 0)

        pltpu.make_async_copy(ystage.at[src_v.at[k]], rows[b], gs[b]).wait()
        pltpu.async_copy(rows[b], acc.at[dst_v.at[k]], ss[b], add=True)
        if with_deg:
          @pl.when(jnp.equal(k < K // 2, cid == 0))
          def _deg():
            pltpu.async_copy(ones_v, dacc.at[dst_v.at[k]], dsem, add=True)
      return carry

    lax.fori_loop(0, K // NB, ring, 0)
    for b in range(NB):
      pltpu.make_async_copy(rows[b], acc.at[dst_v.at[0]], ss[b]).wait()
    if with_deg:
      def dwait(i, carry):
        pltpu.make_async_copy(ones_v, dacc.at[dst_v.at[0]], dsem).wait()
        return carry
      lax.fori_loop(0, K // 2, dwait, 0)
    plsc.subcore_barrier()

    out0 = pl.multiple_of(cid * NP + row0, 8)
    pltpu.sync_copy(acc.at[pl.ds(row0, RPT)], out_hbm.at[pl.ds(out0, RPT)])
    if with_deg:
      pltpu.sync_copy(dacc.at[pl.ds(row0, RPT)], odeg_hbm.at[pl.ds(out0, RPT)])

  return pl.kernel(
      body, mesh=mesh, out_type=out_type, scratch_types=scratch,
      compiler_params=pltpu.CompilerParams(use_tc_tiling_on_sc=False))


_SEGSUM_1 = _make_segsum(D_HID // 2, True)
_SEGSUM_2 = _make_segsum(D_OUT // 2, False)


def _mmy_body(x_ref, w_ref, ya_ref, yb_ref):
  y = jnp.dot(x_ref[...], w_ref[...].astype(jnp.bfloat16),
              preferred_element_type=jnp.float32).astype(jnp.bfloat16)
  dh = y.shape[1] // 2
  ya_ref[...] = y[:, :dh]
  yb_ref[...] = y[:, dh:]


def _mm_y(x, w):
  d_in = x.shape[1]
  d = w.shape[1]
  dh = d // 2
  return pl.pallas_call(
      _mmy_body,
      grid=(NPB,),
      in_specs=[
          pl.BlockSpec((BR, d_in), lambda i: (i, 0)),
          pl.BlockSpec((d_in, d), lambda i: (0, 0)),
      ],
      out_specs=[pl.BlockSpec((BR, dh), lambda i: (i, 0))] * 2,
      out_shape=[jax.ShapeDtypeStruct((NP, dh), jnp.bfloat16)] * 2,
  )(x, w)


def _mmz_body(x_ref, w_ref, b_ref, z_ref):
  xv = x_ref[...]
  z_ref[...] = jnp.dot(xv, w_ref[...].astype(xv.dtype),
                       preferred_element_type=jnp.float32) + b_ref[...]


def _mm_z(x, w, b):
  d_in = x.shape[1]
  d = w.shape[1]
  return pl.pallas_call(
      _mmz_body,
      grid=(NPB,),
      in_specs=[
          pl.BlockSpec((BR, d_in), lambda i: (i, 0)),
          pl.BlockSpec((d_in, d), lambda i: (0, 0)),
          pl.BlockSpec((1, d), lambda i: (0, 0)),
      ],
      out_specs=pl.BlockSpec((BR, d), lambda i: (i, 0)),
      out_shape=jax.ShapeDtypeStruct((NP, d), jnp.float32),
  )(x, w, b.reshape(1, d))


def _comb_body(p0_ref, p1_ref, d0_ref, d1_ref, z1_ref, w_ref,
               h_ref, ya_ref, yb_ref):
  inv = 1.0 / jnp.maximum(d0_ref[...] + d1_ref[...], 1.0)
  agg = jnp.concatenate([p0_ref[...], p1_ref[...]],
                        axis=1).astype(jnp.float32)
  h = jnp.maximum(agg * inv + z1_ref[...], 0.0)
  h_ref[...] = h.astype(jnp.bfloat16)
  y = jnp.dot(h.astype(jnp.bfloat16), w_ref[...].astype(jnp.bfloat16),
              preferred_element_type=jnp.float32).astype(jnp.bfloat16)
  dh = y.shape[1] // 2
  ya_ref[...] = y[:, :dh]
  yb_ref[...] = y[:, dh:]


def _combine_y(p, d0, d1, z1, w):
  dp = p.shape[1]
  d_in = w.shape[0]
  d = w.shape[1]
  dh = d // 2
  return pl.pallas_call(
      _comb_body,
      grid=(NPB,),
      in_specs=[
          pl.BlockSpec((BR, dp), lambda i: (i, 0)),
          pl.BlockSpec((BR, dp), lambda i: (NPB + i, 0)),
          pl.BlockSpec((BR, 1), lambda i: (i, 0)),
          pl.BlockSpec((BR, 1), lambda i: (i, 0)),
          pl.BlockSpec((BR, d_in), lambda i: (i, 0)),
          pl.BlockSpec((d_in, d), lambda i: (0, 0)),
      ],
      out_specs=[
          pl.BlockSpec((BR, d_in), lambda i: (i, 0)),
          pl.BlockSpec((BR, dh), lambda i: (i, 0)),
          pl.BlockSpec((BR, dh), lambda i: (i, 0)),
      ],
      out_shape=[
          jax.ShapeDtypeStruct((NP, d_in), jnp.bfloat16),
          jax.ShapeDtypeStruct((NP, dh), jnp.bfloat16),
          jax.ShapeDtypeStruct((NP, dh), jnp.bfloat16),
      ],
  )(p, p, d0, d1, z1, w)


def _final_body(q0_ref, q1_ref, d0_ref, d1_ref, z2_ref, o_ref):
  inv = 1.0 / jnp.maximum(d0_ref[...] + d1_ref[...], 1.0)
  agg = jnp.concatenate([q0_ref[...], q1_ref[...]],
                        axis=1).astype(jnp.float32)
  o_ref[...] = agg * inv + z2_ref[...]


def _final(q, d0, d1, z2):
  dq = q.shape[1]
  return pl.pallas_call(
      _final_body,
      grid=(NPB,),
      in_specs=[
          pl.BlockSpec((BR, dq), lambda i: (i, 0)),
          pl.BlockSpec((BR, dq), lambda i: (NPB + i, 0)),
          pl.BlockSpec((BR, 1), lambda i: (i, 0)),
          pl.BlockSpec((BR, 1), lambda i: (i, 0)),
          pl.BlockSpec((BR, dq * 2), lambda i: (i, 0)),
      ],
      out_specs=pl.BlockSpec((BR, dq * 2), lambda i: (i, 0)),
      out_shape=jax.ShapeDtypeStruct((NP, dq * 2), jnp.float32),
  )(q, q, d0, d1, z2)


def kernel(x, edge_index, W1_l, W1_r, b1, W2_l, W2_r, b2):
  x = x.astype(jnp.bfloat16)
  src = edge_index[0].astype(jnp.int32)
  dst = edge_index[1].astype(jnp.int32)

  pad = EPAD - E
  src_pp = jnp.concatenate([src, jnp.zeros((pad,), jnp.int32)])
  src_pp = src_pp.reshape(16, K, C)
  pad_dst = N + (jnp.arange(pad, dtype=jnp.int32) % (NP - N))
  dst_b = jnp.concatenate([dst, pad_dst]).reshape(16, K, C)

  x_pad = jnp.pad(x, ((0, NP - N), (0, 0)))
  zrow_1 = jnp.zeros((RPT, D_HID // 2), jnp.bfloat16)
  zrow_2 = jnp.zeros((RPT, D_OUT // 2), jnp.bfloat16)
  zdeg = jnp.zeros((RPT,), jnp.float32)

  y1a, y1b = _mm_y(x_pad, W1_l)
  z1 = _mm_z(x_pad, W1_r, b1)
  p1, dg = _SEGSUM_1(y1a, y1b, src_pp, dst_b, zrow_1, zdeg)
  d0 = dg[:NP].reshape(NP, 1)
  d1 = dg[NP:].reshape(NP, 1)
  h, y2a, y2b = _combine_y(p1, d0, d1, z1, W2_l)
  z2 = _mm_z(h, W2_r, b2)
  (p2,) = _SEGSUM_2(y2a, y2b, src_pp, dst_b, zrow_2)
  out = _final(p2, d0, d1, z2)
  return out[:N]


# speedup vs baseline: 1.1677x; 1.1677x over previous
import jax
import jax.numpy as jnp
from jax import lax
from jax.experimental import pallas as pl
from jax.experimental.pallas import tpu as pltpu
from jax.experimental.pallas import tpu_sc as plsc

N = 10000
E = 320000
D_IN = 128
D_HID = 128
D_OUT = 64

NP = 10240
RPT = NP // 16
C = 128
K = 160
EPT = C * K
EPAD = 16 * EPT

BR = 1024
NPB = NP // BR


def _make_segsum(d_half, with_deg):
  mesh = plsc.VectorSubcoreMesh(core_axis_name="c", subcore_axis_name="s")
  out_type = [jax.ShapeDtypeStruct((2 * NP, d_half), jnp.bfloat16)]
  if with_deg:
    out_type.append(jax.ShapeDtypeStruct((2 * NP,), jnp.float32))
  NB = 8
  G = 6
  scratch = [
      pltpu.VMEM((K, C), jnp.int32),
      pltpu.VMEM((K, C), jnp.int32),
      pltpu.VMEM((NB * C, d_half), jnp.bfloat16),
      pltpu.VMEM_SHARED((NP, d_half), jnp.bfloat16),
      pltpu.VMEM_SHARED((NP, d_half), jnp.bfloat16),
  ] + [pltpu.SemaphoreType.DMA] * (2 * NB)
  if with_deg:
    scratch += [
        pltpu.VMEM((C,), jnp.float32),
        pltpu.VMEM_SHARED((NP,), jnp.float32),
        pltpu.SemaphoreType.DMA,
    ]

  def body(ya_hbm, yb_hbm, src_hbm, dst_hbm, zrow_hbm, *rest):
    if with_deg:
      (zdeg_hbm, out_hbm, odeg_hbm, src_v, dst_v, ring, acc, ystage) = rest[:8]
      rest = rest[8:]
    else:
      (out_hbm, src_v, dst_v, ring, acc, ystage) = rest[:6]
      rest = rest[6:]
    gs = rest[:NB]
    ss = rest[NB:2 * NB]
    rest = rest[2 * NB:]
    if with_deg:
      ones_v, dacc, dsem = rest
    rows = [ring.at[pl.ds(b * C, C)] for b in range(NB)]
    cid = lax.axis_index("c")
    sid = lax.axis_index("s")
    row0 = pl.multiple_of(sid * RPT, 8)

    pltpu.sync_copy(src_hbm.at[sid], src_v)
    pltpu.sync_copy(dst_hbm.at[sid], dst_v)
    pltpu.sync_copy(zrow_hbm, acc.at[pl.ds(row0, RPT)])

    @pl.when(cid == 0)
    def _stage_a():
      pltpu.sync_copy(ya_hbm.at[pl.ds(row0, RPT)], ystage.at[pl.ds(row0, RPT)])

    @pl.when(cid == 1)
    def _stage_b():
      pltpu.sync_copy(yb_hbm.at[pl.ds(row0, RPT)], ystage.at[pl.ds(row0, RPT)])
    if with_deg:
      for i in range(C // 16):
        ones_v[pl.ds(i * 16, 16)] = jnp.ones((16,), jnp.float32)
      pltpu.sync_copy(zdeg_hbm, dacc.at[pl.ds(row0, RPT)])
    plsc.subcore_barrier()

    for k in range(G):
      pltpu.async_copy(ystage.at[src_v.at[k]], rows[k % NB], gs[k % NB])

    def ring(g, carry):
      k0 = g * NB
      for b in range(NB):
        k = k0 + b
        bg = (b + G) % NB

        @pl.when(k + G < K)
        def _refill():
          @pl.when(k >= NB - G)
          def _reclaim():
            pltpu.make_async_copy(rows[bg], acc.at[dst_v.at[0]],
                                  ss[bg]).wait()
          pltpu.async_copy(ystage.at[src_v.at[k + G]], rows[bg], gs[bg])

        pltpu.make_async_copy(ystage.at[src_v.at[k]], rows[b], gs[b]).wait()
        pltpu.async_copy(rows[b], acc.at[dst_v.at[k]], ss[b], add=True)
        if with_deg:
          @pl.when(jnp.equal(k < K // 2, cid == 0))
          def _deg():
            pltpu.async_copy(ones_v, dacc.at[dst_v.at[k]], dsem, add=True)
      return carry

    lax.fori_loop(0, K // NB, ring, 0)
    for b in range(NB):
      pltpu.make_async_copy(rows[b], acc.at[dst_v.at[0]], ss[b]).wait()
    if with_deg:
      def dwait(i, carry):
        pltpu.make_async_copy(ones_v, dacc.at[dst_v.at[0]], dsem).wait()
        return carry
      lax.fori_loop(0, K // 2, dwait, 0)
    plsc.subcore_barrier()

    out0 = pl.multiple_of(cid * NP + row0, 8)
    pltpu.sync_copy(acc.at[pl.ds(row0, RPT)], out_hbm.at[pl.ds(out0, RPT)])
    if with_deg:
      pltpu.sync_copy(dacc.at[pl.ds(row0, RPT)], odeg_hbm.at[pl.ds(out0, RPT)])

  return pl.kernel(
      body, mesh=mesh, out_type=out_type, scratch_types=scratch,
      compiler_params=pltpu.CompilerParams(use_tc_tiling_on_sc=False))


_SEGSUM_1 = _make_segsum(D_HID // 2, True)
_SEGSUM_2 = _make_segsum(D_OUT // 2, False)


def _mmy_body(x_ref, w_ref, ya_ref, yb_ref):
  y = jnp.dot(x_ref[...], w_ref[...].astype(jnp.bfloat16),
              preferred_element_type=jnp.float32).astype(jnp.bfloat16)
  dh = y.shape[1] // 2
  ya_ref[...] = y[:, :dh]
  yb_ref[...] = y[:, dh:]


def _mm_y(x, w):
  d_in = x.shape[1]
  d = w.shape[1]
  dh = d // 2
  return pl.pallas_call(
      _mmy_body,
      grid=(NPB,),
      in_specs=[
          pl.BlockSpec((BR, d_in), lambda i: (i, 0)),
          pl.BlockSpec((d_in, d), lambda i: (0, 0)),
      ],
      out_specs=[pl.BlockSpec((BR, dh), lambda i: (i, 0))] * 2,
      out_shape=[jax.ShapeDtypeStruct((NP, dh), jnp.bfloat16)] * 2,
  )(x, w)


def _mmz_body(x_ref, w_ref, b_ref, z_ref):
  xv = x_ref[...]
  z_ref[...] = jnp.dot(xv, w_ref[...].astype(xv.dtype),
                       preferred_element_type=jnp.float32) + b_ref[...]


def _mm_z(x, w, b):
  d_in = x.shape[1]
  d = w.shape[1]
  return pl.pallas_call(
      _mmz_body,
      grid=(NPB,),
      in_specs=[
          pl.BlockSpec((BR, d_in), lambda i: (i, 0)),
          pl.BlockSpec((d_in, d), lambda i: (0, 0)),
          pl.BlockSpec((1, d), lambda i: (0, 0)),
      ],
      out_specs=pl.BlockSpec((BR, d), lambda i: (i, 0)),
      out_shape=jax.ShapeDtypeStruct((NP, d), jnp.float32),
  )(x, w, b.reshape(1, d))


def _comb_body(p0_ref, p1_ref, d0_ref, d1_ref, z1_ref, w_ref,
               h_ref, ya_ref, yb_ref):
  inv = 1.0 / jnp.maximum(d0_ref[...] + d1_ref[...], 1.0)
  agg = jnp.concatenate([p0_ref[...], p1_ref[...]],
                        axis=1).astype(jnp.float32)
  h = jnp.maximum(agg * inv + z1_ref[...], 0.0)
  h_ref[...] = h.astype(jnp.bfloat16)
  y = jnp.dot(h.astype(jnp.bfloat16), w_ref[...].astype(jnp.bfloat16),
              preferred_element_type=jnp.float32).astype(jnp.bfloat16)
  dh = y.shape[1] // 2
  ya_ref[...] = y[:, :dh]
  yb_ref[...] = y[:, dh:]


def _combine_y(p, d0, d1, z1, w):
  dp = p.shape[1]
  d_in = w.shape[0]
  d = w.shape[1]
  dh = d // 2
  return pl.pallas_call(
      _comb_body,
      grid=(NPB,),
      in_specs=[
          pl.BlockSpec((BR, dp), lambda i: (i, 0)),
          pl.BlockSpec((BR, dp), lambda i: (NPB + i, 0)),
          pl.BlockSpec((BR, 1), lambda i: (i, 0)),
          pl.BlockSpec((BR, 1), lambda i: (i, 0)),
          pl.BlockSpec((BR, d_in), lambda i: (i, 0)),
          pl.BlockSpec((d_in, d), lambda i: (0, 0)),
      ],
      out_specs=[
          pl.BlockSpec((BR, d_in), lambda i: (i, 0)),
          pl.BlockSpec((BR, dh), lambda i: (i, 0)),
          pl.BlockSpec((BR, dh), lambda i: (i, 0)),
      ],
      out_shape=[
          jax.ShapeDtypeStruct((NP, d_in), jnp.bfloat16),
          jax.ShapeDtypeStruct((NP, dh), jnp.bfloat16),
          jax.ShapeDtypeStruct((NP, dh), jnp.bfloat16),
      ],
  )(p, p, d0, d1, z1, w)


def _final_body(q0_ref, q1_ref, d0_ref, d1_ref, z2_ref, o_ref):
  inv = 1.0 / jnp.maximum(d0_ref[...] + d1_ref[...], 1.0)
  agg = jnp.concatenate([q0_ref[...], q1_ref[...]],
                        axis=1).astype(jnp.float32)
  o_ref[...] = agg * inv + z2_ref[...]


def _final(q, d0, d1, z2):
  dq = q.shape[1]
  return pl.pallas_call(
      _final_body,
      grid=(NPB,),
      in_specs=[
          pl.BlockSpec((BR, dq), lambda i: (i, 0)),
          pl.BlockSpec((BR, dq), lambda i: (NPB + i, 0)),
          pl.BlockSpec((BR, 1), lambda i: (i, 0)),
          pl.BlockSpec((BR, 1), lambda i: (i, 0)),
          pl.BlockSpec((BR, dq * 2), lambda i: (i, 0)),
      ],
      out_specs=pl.BlockSpec((BR, dq * 2), lambda i: (i, 0)),
      out_shape=jax.ShapeDtypeStruct((NP, dq * 2), jnp.float32),
  )(q, q, d0, d1, z2)


def kernel(x, edge_index, W1_l, W1_r, b1, W2_l, W2_r, b2):
  x = x.astype(jnp.bfloat16)
  src = edge_index[0].astype(jnp.int32)
  dst = edge_index[1].astype(jnp.int32)

  pad = EPAD - E
  src_pp = jnp.concatenate([src, jnp.zeros((pad,), jnp.int32)])
  src_pp = src_pp.reshape(16, K, C)
  pad_dst = N + (jnp.arange(pad, dtype=jnp.int32) % (NP - N))
  dst_b = jnp.concatenate([dst, pad_dst]).reshape(16, K, C)

  x_pad = jnp.pad(x, ((0, NP - N), (0, 0)))
  zrow_1 = jnp.zeros((RPT, D_HID // 2), jnp.bfloat16)
  zrow_2 = jnp.zeros((RPT, D_OUT // 2), jnp.bfloat16)
  zdeg = jnp.zeros((RPT,), jnp.float32)

  y1a, y1b = _mm_y(x_pad, W1_l)
  z1 = _mm_z(x_pad, W1_r, b1)
  p1, dg = _SEGSUM_1(y1a, y1b, src_pp, dst_b, zrow_1, zdeg)
  d0 = dg[:NP].reshape(NP, 1)
  d1 = dg[NP:].reshape(NP, 1)
  h, y2a, y2b = _combine_y(p1, d0, d1, z1, W2_l)
  z2 = _mm_z(h, W2_r, b2)
  (p2,) = _SEGSUM_2(y2a, y2b, src_pp, dst_b, zrow_2)
  out = _final(p2, d0, d1, z2)
  return out[:N]


# speedup vs baseline: 1.2011x; 1.0286x over previous
import jax
import jax.numpy as jnp
from jax import lax
from jax.experimental import pallas as pl
from jax.experimental.pallas import tpu as pltpu
from jax.experimental.pallas import tpu_sc as plsc

N = 10000
E = 320000
D_IN = 128
D_HID = 128
D_OUT = 64

NP = 10240
RPT = NP // 16
C = 128
K = 160
EPT = C * K
EPAD = 16 * EPT

BR = 1024
NPB = NP // BR


def _make_segsum(d_half, with_deg):
  mesh = plsc.VectorSubcoreMesh(core_axis_name="c", subcore_axis_name="s")
  out_type = [jax.ShapeDtypeStruct((2 * NP, d_half), jnp.bfloat16)]
  if with_deg:
    out_type.append(jax.ShapeDtypeStruct((2 * NP,), jnp.float32))
  NB = 8
  G = 5
  scratch = [
      pltpu.VMEM((K, C), jnp.int32),
      pltpu.VMEM((K, C), jnp.int32),
      pltpu.VMEM((NB * C, d_half), jnp.bfloat16),
      pltpu.VMEM_SHARED((NP, d_half), jnp.bfloat16),
      pltpu.VMEM_SHARED((NP, d_half), jnp.bfloat16),
  ] + [pltpu.SemaphoreType.DMA] * (2 * NB)
  if with_deg:
    scratch += [
        pltpu.VMEM((C,), jnp.float32),
        pltpu.VMEM_SHARED((NP,), jnp.float32),
        pltpu.SemaphoreType.DMA,
    ]

  def body(ya_hbm, yb_hbm, src_hbm, dst_hbm, zrow_hbm, *rest):
    if with_deg:
      (zdeg_hbm, out_hbm, odeg_hbm, src_v, dst_v, ring, acc, ystage) = rest[:8]
      rest = rest[8:]
    else:
      (out_hbm, src_v, dst_v, ring, acc, ystage) = rest[:6]
      rest = rest[6:]
    gs = rest[:NB]
    ss = rest[NB:2 * NB]
    rest = rest[2 * NB:]
    if with_deg:
      ones_v, dacc, dsem = rest
    rows = [ring.at[pl.ds(b * C, C)] for b in range(NB)]
    cid = lax.axis_index("c")
    sid = lax.axis_index("s")
    row0 = pl.multiple_of(sid * RPT, 8)

    cp0 = pltpu.async_copy(src_hbm.at[sid], src_v, gs[0])
    cp1 = pltpu.async_copy(dst_hbm.at[sid], dst_v, gs[1])
    cp2 = pltpu.async_copy(zrow_hbm, acc.at[pl.ds(row0, RPT)], gs[2])

    @pl.when(cid == 0)
    def _stage_a():
      pltpu.async_copy(ya_hbm.at[pl.ds(row0, RPT)],
                       ystage.at[pl.ds(row0, RPT)], gs[3])

    @pl.when(cid == 1)
    def _stage_b():
      pltpu.async_copy(yb_hbm.at[pl.ds(row0, RPT)],
                       ystage.at[pl.ds(row0, RPT)], gs[3])
    if with_deg:
      for i in range(C // 16):
        ones_v[pl.ds(i * 16, 16)] = jnp.ones((16,), jnp.float32)
      pltpu.sync_copy(zdeg_hbm, dacc.at[pl.ds(row0, RPT)])
    cp0.wait()
    cp1.wait()
    cp2.wait()
    pltpu.make_async_copy(ya_hbm.at[pl.ds(row0, RPT)],
                          ystage.at[pl.ds(row0, RPT)], gs[3]).wait()
    plsc.subcore_barrier()

    for k in range(G):
      pltpu.async_copy(ystage.at[src_v.at[k]], rows[k % NB], gs[k % NB])

    def ring(g, carry):
      k0 = g * NB
      for b in range(NB):
        k = k0 + b
        bg = (b + G) % NB

        @pl.when(k + G < K)
        def _refill():
          @pl.when(k >= NB - G)
          def _reclaim():
            pltpu.make_async_copy(rows[bg], acc.at[dst_v.at[0]],
                                  ss[bg]).wait()
          pltpu.async_copy(ystage.at[src_v.at[k + G]], rows[bg], gs[bg])

        pltpu.make_async_copy(ystage.at[src_v.at[k]], rows[b], gs[b]).wait()
        pltpu.async_copy(rows[b], acc.at[dst_v.at[k]], ss[b], add=True)
        if with_deg:
          @pl.when(jnp.equal(k < K // 2, cid == 0))
          def _deg():
            pltpu.async_copy(ones_v, dacc.at[dst_v.at[k]], dsem, add=True)
      return carry

    lax.fori_loop(0, K // NB, ring, 0)
    for b in range(NB):
      pltpu.make_async_copy(rows[b], acc.at[dst_v.at[0]], ss[b]).wait()
    if with_deg:
      def dwait(i, carry):
        pltpu.make_async_copy(ones_v, dacc.at[dst_v.at[0]], dsem).wait()
        return carry
      lax.fori_loop(0, K // 2, dwait, 0)
    plsc.subcore_barrier()

    out0 = pl.multiple_of(cid * NP + row0, 8)
    pltpu.sync_copy(acc.at[pl.ds(row0, RPT)], out_hbm.at[pl.ds(out0, RPT)])
    if with_deg:
      pltpu.sync_copy(dacc.at[pl.ds(row0, RPT)], odeg_hbm.at[pl.ds(out0, RPT)])

  return pl.kernel(
      body, mesh=mesh, out_type=out_type, scratch_types=scratch,
      compiler_params=pltpu.CompilerParams(use_tc_tiling_on_sc=False))


_SEGSUM_1 = _make_segsum(D_HID // 2, True)
_SEGSUM_2 = _make_segsum(D_OUT // 2, False)


def _mmy_body(x_ref, w_ref, ya_ref, yb_ref):
  y = jnp.dot(x_ref[...], w_ref[...].astype(jnp.bfloat16),
              preferred_element_type=jnp.float32).astype(jnp.bfloat16)
  dh = y.shape[1] // 2
  ya_ref[...] = y[:, :dh]
  yb_ref[...] = y[:, dh:]


def _mm_y(x, w):
  d_in = x.shape[1]
  d = w.shape[1]
  dh = d // 2
  return pl.pallas_call(
      _mmy_body,
      grid=(NPB,),
      in_specs=[
          pl.BlockSpec((BR, d_in), lambda i: (i, 0)),
          pl.BlockSpec((d_in, d), lambda i: (0, 0)),
      ],
      out_specs=[pl.BlockSpec((BR, dh), lambda i: (i, 0))] * 2,
      out_shape=[jax.ShapeDtypeStruct((NP, dh), jnp.bfloat16)] * 2,
  )(x, w)


def _mmz_body(x_ref, w_ref, b_ref, z_ref):
  xv = x_ref[...]
  z_ref[...] = jnp.dot(xv, w_ref[...].astype(xv.dtype),
                       preferred_element_type=jnp.float32) + b_ref[...]


def _mm_z(x, w, b):
  d_in = x.shape[1]
  d = w.shape[1]
  return pl.pallas_call(
      _mmz_body,
      grid=(NPB,),
      in_specs=[
          pl.BlockSpec((BR, d_in), lambda i: (i, 0)),
          pl.BlockSpec((d_in, d), lambda i: (0, 0)),
          pl.BlockSpec((1, d), lambda i: (0, 0)),
      ],
      out_specs=pl.BlockSpec((BR, d), lambda i: (i, 0)),
      out_shape=jax.ShapeDtypeStruct((NP, d), jnp.float32),
  )(x, w, b.reshape(1, d))


def _comb_body(p0_ref, p1_ref, d0_ref, d1_ref, z1_ref, w_ref,
               h_ref, ya_ref, yb_ref):
  inv = 1.0 / jnp.maximum(d0_ref[...] + d1_ref[...], 1.0)
  agg = jnp.concatenate([p0_ref[...], p1_ref[...]],
                        axis=1).astype(jnp.float32)
  h = jnp.maximum(agg * inv + z1_ref[...], 0.0)
  h_ref[...] = h.astype(jnp.bfloat16)
  y = jnp.dot(h.astype(jnp.bfloat16), w_ref[...].astype(jnp.bfloat16),
              preferred_element_type=jnp.float32).astype(jnp.bfloat16)
  dh = y.shape[1] // 2
  ya_ref[...] = y[:, :dh]
  yb_ref[...] = y[:, dh:]


def _combine_y(p, d0, d1, z1, w):
  dp = p.shape[1]
  d_in = w.shape[0]
  d = w.shape[1]
  dh = d // 2
  return pl.pallas_call(
      _comb_body,
      grid=(NPB,),
      in_specs=[
          pl.BlockSpec((BR, dp), lambda i: (i, 0)),
          pl.BlockSpec((BR, dp), lambda i: (NPB + i, 0)),
          pl.BlockSpec((BR, 1), lambda i: (i, 0)),
          pl.BlockSpec((BR, 1), lambda i: (i, 0)),
          pl.BlockSpec((BR, d_in), lambda i: (i, 0)),
          pl.BlockSpec((d_in, d), lambda i: (0, 0)),
      ],
      out_specs=[
          pl.BlockSpec((BR, d_in), lambda i: (i, 0)),
          pl.BlockSpec((BR, dh), lambda i: (i, 0)),
          pl.BlockSpec((BR, dh), lambda i: (i, 0)),
      ],
      out_shape=[
          jax.ShapeDtypeStruct((NP, d_in), jnp.bfloat16),
          jax.ShapeDtypeStruct((NP, dh), jnp.bfloat16),
          jax.ShapeDtypeStruct((NP, dh), jnp.bfloat16),
      ],
  )(p, p, d0, d1, z1, w)


def _final_body(q0_ref, q1_ref, d0_ref, d1_ref, z2_ref, o_ref):
  inv = 1.0 / jnp.maximum(d0_ref[...] + d1_ref[...], 1.0)
  agg = jnp.concatenate([q0_ref[...], q1_ref[...]],
                        axis=1).astype(jnp.float32)
  o_ref[...] = agg * inv + z2_ref[...]


def _final(q, d0, d1, z2):
  dq = q.shape[1]
  return pl.pallas_call(
      _final_body,
      grid=(NPB,),
      in_specs=[
          pl.BlockSpec((BR, dq), lambda i: (i, 0)),
          pl.BlockSpec((BR, dq), lambda i: (NPB + i, 0)),
          pl.BlockSpec((BR, 1), lambda i: (i, 0)),
          pl.BlockSpec((BR, 1), lambda i: (i, 0)),
          pl.BlockSpec((BR, dq * 2), lambda i: (i, 0)),
      ],
      out_specs=pl.BlockSpec((BR, dq * 2), lambda i: (i, 0)),
      out_shape=jax.ShapeDtypeStruct((NP, dq * 2), jnp.float32),
  )(q, q, d0, d1, z2)


def kernel(x, edge_index, W1_l, W1_r, b1, W2_l, W2_r, b2):
  x = x.astype(jnp.bfloat16)
  src = edge_index[0].astype(jnp.int32)
  dst = edge_index[1].astype(jnp.int32)

  pad = EPAD - E
  src_pp = jnp.concatenate([src, jnp.zeros((pad,), jnp.int32)])
  src_pp = src_pp.reshape(16, K, C)
  pad_dst = N + (jnp.arange(pad, dtype=jnp.int32) % (NP - N))
  dst_b = jnp.concatenate([dst, pad_dst]).reshape(16, K, C)

  x_pad = jnp.pad(x, ((0, NP - N), (0, 0)))
  zrow_1 = jnp.zeros((RPT, D_HID // 2), jnp.bfloat16)
  zrow_2 = jnp.zeros((RPT, D_OUT // 2), jnp.bfloat16)
  zdeg = jnp.zeros((RPT,), jnp.float32)

  y1a, y1b = _mm_y(x_pad, W1_l)
  z1 = _mm_z(x_pad, W1_r, b1)
  p1, dg = _SEGSUM_1(y1a, y1b, src_pp, dst_b, zrow_1, zdeg)
  d0 = dg[:NP].reshape(NP, 1)
  d1 = dg[NP:].reshape(NP, 1)
  h, y2a, y2b = _combine_y(p1, d0, d1, z1, W2_l)
  z2 = _mm_z(h, W2_r, b2)
  (p2,) = _SEGSUM_2(y2a, y2b, src_pp, dst_b, zrow_2)
  out = _final(p2, d0, d1, z2)
  return out[:N]
